# baseline (speedup 1.0000x reference)
import functools

import jax
import jax.numpy as jnp
from jax import lax
from jax.experimental import pallas as pl
from jax.experimental.pallas import tpu as pltpu
from jax.experimental.pallas import tpu_sc as plsc

VOCAB = 100000
EMBED_DIM = 32
HIDDEN_DIM = 128
BATCH = 1024
CTX = 5

NUM_IDX = BATCH * CTX
SC_NC, SC_NS = 2, 16
SC_NW = SC_NC * SC_NS
ROWS_PER_W = NUM_IDX // SC_NW

OUT_TILE = 2048


def _sc_gather(emb, idx_flat):
    mesh = plsc.VectorSubcoreMesh(core_axis_name="c", subcore_axis_name="s")

    @functools.partial(
        pl.kernel,
        mesh=mesh,
        out_type=jax.ShapeDtypeStruct((NUM_IDX, EMBED_DIM), jnp.float32),
        scratch_types=[
            pltpu.VMEM((ROWS_PER_W,), jnp.int32),
            pltpu.VMEM((ROWS_PER_W, EMBED_DIM), jnp.float32),
            pltpu.SemaphoreType.DMA,
        ],
        compiler_params=pltpu.CompilerParams(use_tc_tiling_on_sc=False),
    )
    def gather_kernel(table_hbm, idx_hbm, out_hbm, idx_v, rows_v, sem):
        wid = lax.axis_index("s") * SC_NC + lax.axis_index("c")
        base = wid * ROWS_PER_W
        pltpu.sync_copy(idx_hbm.at[pl.ds(base, ROWS_PER_W)], idx_v)
        pltpu.async_copy(table_hbm.at[idx_v], rows_v, sem).wait()
        pltpu.sync_copy(rows_v, out_hbm.at[pl.ds(base, ROWS_PER_W)])

    return gather_kernel(emb, idx_flat)


def _hidden_body(cv_ref, wh_ref, bh_ref, h_ref):
    h = jnp.dot(cv_ref[...], wh_ref[...], preferred_element_type=jnp.float32)
    h_ref[...] = jnp.maximum(h + bh_ref[...], 0.0).astype(jnp.bfloat16)


def _out_body(h_ref, wo_ref, bo_ref, o_ref):
    wo = wo_ref[...].astype(jnp.bfloat16)
    o = jnp.dot(h_ref[...], wo, preferred_element_type=jnp.float32)
    o_ref[...] = o + bo_ref[...]


def kernel(context, emb, W_h, b_h, W_o, b_o):
    idx_flat = context.reshape(NUM_IDX).astype(jnp.int32)
    gathered = _sc_gather(emb, idx_flat)
    cv = gathered.reshape(BATCH, CTX * EMBED_DIM)

    hidden = pl.pallas_call(
        _hidden_body,
        out_shape=jax.ShapeDtypeStruct((BATCH, HIDDEN_DIM), jnp.bfloat16),
    )(cv, W_h, b_h.reshape(1, HIDDEN_DIM))

    n_out = W_o.shape[1]
    grid = pl.cdiv(n_out, OUT_TILE)
    out = pl.pallas_call(
        _out_body,
        grid=(grid,),
        in_specs=[
            pl.BlockSpec((BATCH, HIDDEN_DIM), lambda i: (0, 0)),
            pl.BlockSpec((HIDDEN_DIM, OUT_TILE), lambda i: (0, i)),
            pl.BlockSpec((1, OUT_TILE), lambda i: (0, i)),
        ],
        out_specs=pl.BlockSpec((BATCH, OUT_TILE), lambda i: (0, i)),
        out_shape=jax.ShapeDtypeStruct((BATCH, n_out), jnp.float32),
        compiler_params=pltpu.CompilerParams(
            dimension_semantics=("parallel",),
        ),
    )(hidden, W_o, b_o.reshape(1, n_out))

    return out.reshape(BATCH, 3, VOCAB)


# speedup vs baseline: 3.5987x; 3.5987x over previous
import functools

import jax
import jax.numpy as jnp
from jax import lax
from jax.experimental import pallas as pl
from jax.experimental.pallas import tpu as pltpu
from jax.experimental.pallas import tpu_sc as plsc

VOCAB = 100000
EMBED_DIM = 32
HIDDEN_DIM = 128
BATCH = 1024
CTX = 5

NUM_IDX = BATCH * CTX
SC_NC, SC_NS = 2, 16
SC_NW = SC_NC * SC_NS
ROWS_PER_W = NUM_IDX // SC_NW

WIDEN_W = 12800
OUT_TILE = 5000


def _widen_body(et_ref, t4_ref):
    y = et_ref[...].T
    z = jnp.zeros((WIDEN_W, 128 - EMBED_DIM), jnp.float32)
    t4_ref[...] = jnp.concatenate([y, z], axis=1)


def _widen(emb_t):
    return pl.pallas_call(
        _widen_body,
        grid=(pl.cdiv(VOCAB, WIDEN_W),),
        in_specs=[pl.BlockSpec((EMBED_DIM, WIDEN_W), lambda i: (0, i))],
        out_specs=pl.BlockSpec((WIDEN_W, 128), lambda i: (i, 0)),
        out_shape=jax.ShapeDtypeStruct((VOCAB, 128), jnp.float32),
    )(emb_t)


def _sc_gather(t4, idx_flat):
    mesh = plsc.VectorSubcoreMesh(core_axis_name="c", subcore_axis_name="s")

    @functools.partial(
        pl.kernel,
        mesh=mesh,
        out_type=jax.ShapeDtypeStruct((SC_NW, ROWS_PER_W, 128), jnp.float32),
        scratch_types=[
            pltpu.VMEM((ROWS_PER_W,), jnp.int32),
            pltpu.VMEM((ROWS_PER_W, 128), jnp.float32),
            pltpu.SemaphoreType.DMA,
        ],
    )
    def gather_kernel(table_hbm, idx_hbm, out_hbm, idx_v, rows_v, sem):
        wid = lax.axis_index("s") * SC_NC + lax.axis_index("c")
        base = wid * ROWS_PER_W
        pltpu.sync_copy(idx_hbm.at[pl.ds(base, ROWS_PER_W)], idx_v)
        pltpu.async_copy(table_hbm.at[idx_v], rows_v, sem).wait()
        pltpu.sync_copy(rows_v, out_hbm.at[wid])

    return gather_kernel(t4, idx_flat)


def _fused_body(cv_ref, wh_ref, bh_ref, wot_ref, bo_ref, ot_ref, ht_ref):
    @pl.when(pl.program_id(0) == 0)
    def _():
        h = jnp.dot(cv_ref[...], wh_ref[...], preferred_element_type=jnp.float32)
        h = jnp.maximum(h + bh_ref[...], 0.0)
        ht_ref[...] = h.T.astype(jnp.bfloat16)

    wot = wot_ref[...].astype(jnp.bfloat16)
    acc = jnp.dot(wot, ht_ref[...], preferred_element_type=jnp.float32)
    ot_ref[...] = acc + bo_ref[0].T


def kernel(context, emb, W_h, b_h, W_o, b_o):
    idx_flat = context.reshape(NUM_IDX).astype(jnp.int32)
    t4 = _widen(emb.T)
    gathered = _sc_gather(t4, idx_flat)
    cv640 = gathered.reshape(BATCH, CTX * 128)

    wh640 = jnp.pad(
        W_h.reshape(CTX, EMBED_DIM, HIDDEN_DIM),
        ((0, 0), (0, 128 - EMBED_DIM), (0, 0)),
    ).reshape(CTX * 128, HIDDEN_DIM)

    n_out = W_o.shape[1]
    grid = n_out // OUT_TILE
    out_t = pl.pallas_call(
        _fused_body,
        grid=(grid,),
        in_specs=[
            pl.BlockSpec((BATCH, CTX * 128), lambda i: (0, 0)),
            pl.BlockSpec((CTX * 128, HIDDEN_DIM), lambda i: (0, 0)),
            pl.BlockSpec((1, HIDDEN_DIM), lambda i: (0, 0)),
            pl.BlockSpec((OUT_TILE, HIDDEN_DIM), lambda i: (i, 0)),
            pl.BlockSpec((1, 1, OUT_TILE), lambda i: (i, 0, 0)),
        ],
        out_specs=pl.BlockSpec((OUT_TILE, BATCH), lambda i: (i, 0)),
        out_shape=jax.ShapeDtypeStruct((n_out, BATCH), jnp.float32),
        scratch_shapes=[pltpu.VMEM((HIDDEN_DIM, BATCH), jnp.bfloat16)],
    )(cv640, wh640, b_h.reshape(1, HIDDEN_DIM), W_o.T,
      b_o.reshape(grid, 1, OUT_TILE))

    return out_t.reshape(3, VOCAB, BATCH).transpose(2, 0, 1)
